# Initial kernel scaffold; baseline (speedup 1.0000x reference)
import jax
import jax.numpy as jnp
from jax.experimental import pallas as pl


def kernel(Z, table):
    raise NotImplementedError("write your pallas kernel here")


# speedup vs baseline: 2.4116x; 2.4116x over previous
import functools

import jax
import jax.numpy as jnp
from jax import lax
from jax.experimental import pallas as pl
from jax.experimental.pallas import tpu as pltpu
from jax.experimental.pallas import tpu_sc as plsc

NUM_CORES = 2
NUM_SUBCORES = 16
NW = NUM_CORES * NUM_SUBCORES
CHUNK = 128
D = 128


def _emb_body(z_hbm, table_hbm, out_hbm, idx_v, rows_v, gsem, ssem):
    wid = lax.axis_index("s") * NUM_CORES + lax.axis_index("c")
    nchunks = z_hbm.shape[1]
    rows_per_w = nchunks * CHUNK
    base = wid * rows_per_w

    pltpu.sync_copy(z_hbm.at[wid], idx_v)

    def body(c, carry):
        pltpu.async_copy(table_hbm.at[idx_v.at[c]], rows_v, gsem).wait()
        pltpu.async_copy(
            rows_v, out_hbm.at[pl.ds(base + c * CHUNK, CHUNK)], ssem
        ).wait()
        return carry

    lax.fori_loop(0, nchunks, body, 0)


def kernel(Z, table):
    B = Z.shape[0] * Z.shape[1]
    n_per_w = B // NW
    nchunks = n_per_w // CHUNK
    z_flat = Z.reshape(NW, nchunks, CHUNK).astype(jnp.int32)
    table = table.at[0].set(0.0)

    mesh = plsc.VectorSubcoreMesh(
        core_axis_name="c", subcore_axis_name="s",
        num_cores=NUM_CORES, num_subcores=NUM_SUBCORES,
    )
    run = pl.kernel(
        _emb_body,
        out_type=jax.ShapeDtypeStruct((B, D), jnp.float32),
        mesh=mesh,
        scratch_types=[
            pltpu.VMEM((nchunks, CHUNK), jnp.int32),
            pltpu.VMEM((CHUNK, D), jnp.float32),
            pltpu.SemaphoreType.DMA,
            pltpu.SemaphoreType.DMA,
        ],
    )
    out = run(z_flat, table)
    return out.reshape(Z.shape[0], Z.shape[1], D)
